# baseline (speedup 1.0000x reference)
import jax
import jax.numpy as jnp
from jax import lax
from jax.experimental import pallas as pl
from jax.experimental.pallas import tpu as pltpu
from jax.experimental.pallas import tpu_sc as plsc

_NC = 2
_NS = 16
_NW = _NC * _NS
_L = 16

_BLK = 128
_NJ = 50
_D = 64
_NBUF = 5


def _emb_body(x_hbm, table_hbm, out_hbm, idx_all, idx_j, rows_v, trans_v,
              *sems):
    gsems = sems[:_NBUF]
    wsems = sems[_NBUF:]
    wid = lax.axis_index("s") * _NC + lax.axis_index("c")

    pltpu.sync_copy(x_hbm.at[wid], idx_all)

    iota = lax.iota(jnp.int32, _L)

    def build_idx(j, b):
        for g in range(_BLK // _L):
            src = (iota + (g * _L)) * _NJ + j
            idx_j[b, g * _L:(g + 1) * _L] = plsc.load_gather(idx_all, [src])

    def gather(b):
        return pltpu.make_async_copy(
            table_hbm.at[idx_j.at[b]], rows_v.at[b], gsems[b]
        )

    def writeback(j, b):
        return pltpu.make_async_copy(
            trans_v.at[b, :, :, pl.ds(0, 128)], out_hbm.at[j, :, wid],
            wsems[b],
        )

    t_vec = [(iota + dg * _L) // 8 for dg in range(_D // _L)]
    r_vec = [(iota + dg * _L) % 8 for dg in range(_D // _L)]

    _TOK = 8
    _LA = 4
    _NG = _TOK * (_D // _L)

    def transpose(b):
        @pl.loop(0, _BLK, step=_TOK)
        def _l(l0):
            lanes = [jnp.full((_L,), l0 + u, jnp.int32) for u in range(_TOK)]
            vals = [None] * _NG
            for k in range(_NG + _LA):
                if k < _NG:
                    u, dg = divmod(k, _D // _L)
                    vals[k] = rows_v[b, l0 + u, dg * _L:(dg + 1) * _L]
                if k >= _LA:
                    u, dg = divmod(k - _LA, _D // _L)
                    plsc.store_scatter(
                        trans_v.at[b],
                        [t_vec[dg], r_vec[dg], lanes[u]],
                        vals[k - _LA],
                    )

    for b in range(_NBUF):
        build_idx(b, b)
        gather(b).start()

    @pl.loop(0, _NJ // _NBUF)
    def _outer(o):
        for b in range(_NBUF):
            j = o * _NBUF + b
            gather(b).wait()

            @pl.when(j >= _NBUF)
            def _():
                writeback(j - _NBUF, b).wait()

            transpose(b)
            writeback(j, b).start()
            nxt = j + _NBUF

            @pl.when(nxt < _NJ)
            def _():
                build_idx(nxt, b)
                gather(b).start()

    for b in range(_NBUF):
        writeback(_NJ - _NBUF + b, b).wait()


@jax.jit
def kernel(x, table):
    B0, B1 = x.shape
    V, D = table.shape

    idx = x.reshape(_NW, _BLK * B1).astype(jnp.int32)

    mesh = plsc.VectorSubcoreMesh(core_axis_name="c", subcore_axis_name="s")
    run = pl.kernel(
        _emb_body,
        out_type=jax.ShapeDtypeStruct((B1, 8, _NW, 8, 128), jnp.float32),
        mesh=mesh,
        scratch_types=[
            pltpu.VMEM((_BLK * B1,), jnp.int32),
            pltpu.VMEM((_NBUF, _BLK), jnp.int32),
            pltpu.VMEM((_NBUF, _BLK, D), jnp.float32),
            pltpu.VMEM((_NBUF, 8, 8, 129), jnp.float32),
        ]
        + [pltpu.SemaphoreType.DMA] * (2 * _NBUF),
        compiler_params=pltpu.CompilerParams(
            use_tc_tiling_on_sc=False, needs_layout_passes=False,
            skip_device_barrier=True,
        ),
        name="sc_embedding_gather",
    )
    out5d = run(idx, table)
    return out5d.transpose(2, 4, 0, 1, 3).reshape(B0, B1, D)


# speedup vs baseline: 1.0007x; 1.0007x over previous
import jax
import jax.numpy as jnp
from jax import lax
from jax.experimental import pallas as pl
from jax.experimental.pallas import tpu as pltpu
from jax.experimental.pallas import tpu_sc as plsc

_NC = 2
_NS = 16
_NW = _NC * _NS
_L = 16

_BLK = 128
_NJ = 50
_D = 64
_NBUF = 5


def _emb_body(x_hbm, table_hbm, out_hbm, idx_all, idx_j, rows_v, trans_v,
              *sems):
    gsems = sems[:_NBUF]
    wsems = sems[_NBUF:]
    wid = lax.axis_index("s") * _NC + lax.axis_index("c")

    pltpu.sync_copy(x_hbm.at[wid], idx_all)

    iota = lax.iota(jnp.int32, _L)

    def build_idx(j, b):
        for g in range(_BLK // _L):
            src = (iota + (g * _L)) * _NJ + j
            idx_j[b, g * _L:(g + 1) * _L] = plsc.load_gather(idx_all, [src])

    def gather(b):
        return pltpu.make_async_copy(
            table_hbm.at[idx_j.at[b]], rows_v.at[b], gsems[b]
        )

    def writeback(j, b):
        return pltpu.make_async_copy(
            trans_v.at[b, :, :, pl.ds(0, 128)], out_hbm.at[j, :, wid],
            wsems[b],
        )

    t_vec = [(iota + dg * _L) // 8 for dg in range(_D // _L)]
    r_vec = [(iota + dg * _L) % 8 for dg in range(_D // _L)]

    _TOK = 8
    _LA = 4
    _NG = _TOK * (_D // _L)

    def transpose(b):
        @pl.loop(0, _BLK, step=_TOK)
        def _l(l0):
            lanes = [jnp.full((_L,), l0 + u, jnp.int32) for u in range(_TOK)]
            vals = [None] * _NG
            for k in range(_NG + _LA):
                if k < _NG:
                    u, dg = divmod(k, _D // _L)
                    vals[k] = rows_v[b, l0 + u, dg * _L:(dg + 1) * _L]
                if k >= _LA:
                    u, dg = divmod(k - _LA, _D // _L)
                    plsc.store_scatter(
                        trans_v.at[b],
                        [t_vec[dg], r_vec[dg], lanes[u]],
                        vals[k - _LA],
                    )

    for b in range(_NBUF):
        build_idx(b, b)
        gather(b).start()

    @pl.loop(0, _NJ // _NBUF)
    def _outer(o):
        for b in range(_NBUF):
            j = o * _NBUF + b
            gather(b).wait()

            @pl.when(j >= _NBUF)
            def _():
                writeback(j - _NBUF, b).wait()

            transpose(b)
            writeback(j, b).start()
            nxt = j + _NBUF

            @pl.when(nxt < _NJ)
            def _():
                build_idx(nxt, b)
                gather(b).start()

    for b in range(_NBUF):
        writeback(_NJ - _NBUF + b, b).wait()


@jax.jit
def kernel(x, table):
    B0, B1 = x.shape
    V, D = table.shape

    idx = x.reshape(_NW, _BLK * B1).astype(jnp.int32)

    mesh = plsc.VectorSubcoreMesh(core_axis_name="c", subcore_axis_name="s")
    run = pl.kernel(
        _emb_body,
        out_type=jax.ShapeDtypeStruct((B1, 8, _NW, 8, 128), jnp.float32),
        mesh=mesh,
        scratch_types=[
            pltpu.VMEM((_BLK * B1,), jnp.int32),
            pltpu.VMEM((_NBUF, _BLK), jnp.int32),
            pltpu.VMEM((_NBUF, _BLK, D), jnp.float32),
            pltpu.VMEM((_NBUF, 8, 8, 129), jnp.float32),
        ]
        + [pltpu.SemaphoreType.DMA] * (2 * _NBUF),
        compiler_params=pltpu.CompilerParams(
            use_tc_tiling_on_sc=False, needs_layout_passes=False
        ),
        name="sc_embedding_gather",
    )
    out5d = run(idx, table)
    return out5d.transpose(2, 4, 0, 1, 3).reshape(B0, B1, D)
